# baseline (speedup 1.0000x reference)
import jax
import jax.numpy as jnp
from jax import lax
from jax.experimental import pallas as pl
from jax.experimental.pallas import tpu as pltpu
from jax.experimental.pallas import tpu_sc as plsc

RES = 2048
CH = 8
B = 4
HW = 512 * 512
NPTS = B * HW
NW = 32

_PARAMS = pltpu.CompilerParams(needs_layout_passes=False,
                               use_tc_tiling_on_sc=False)


def _wid():
    return lax.axis_index("s") * 2 + lax.axis_index("c")


NUNIT = 256 * 16
UPW = NUNIT // NW


def _tr_body(tex4_hbm, out_hbm, in_v, out_v, semi0, semi1, semo0, semo1):
    w = _wid()
    lane = lax.iota(jnp.int32, 16)
    cvec = lane % 8
    dvec = lane // 8

    def fire_in(u, p, sem):
        uid = jnp.minimum(w * UPW + u, NUNIT - 1)
        yt = uid // 16
        xt = uid % 16
        return pltpu.async_copy(tex4_hbm.at[:, yt, xt], in_v.at[p], sem)

    zero = jnp.zeros((16,), jnp.int32)
    flat0 = cvec * 1024 + dvec

    def shuffle(p):
        def shuf(h, idxv, p=p):
            val = plsc.load_gather(in_v.at[p], [zero, idxv])
            out_v[p, h // 64, pl.ds((h % 64) * 16, 16)] = val
            return idxv + 2

        lax.fori_loop(0, 512, shuf, flat0, unroll=8)

    def fire_out(u, p, sem):
        uid = w * UPW + u
        yt = uid // 16
        xt = uid % 16
        return pltpu.async_copy(out_v.at[p],
                                out_hbm.at[pl.ds(yt * 8, 8), xt], sem)

    fire_in(0, 0, semi0)
    fire_in(1, 1, semi1)

    def pair_body(i, carry):
        u0 = 2 * i
        pltpu.make_async_copy(tex4_hbm.at[:, 0, 0], in_v.at[0], semi0).wait()

        @pl.when(i > 0)
        def _():
            pltpu.make_async_copy(out_v.at[0],
                                  out_hbm.at[pl.ds(0, 8), 0], semo0).wait()

        shuffle(0)
        fire_in(u0 + 2, 0, semi0)
        fire_out(u0, 0, semo0)
        pltpu.make_async_copy(tex4_hbm.at[:, 0, 0], in_v.at[1], semi1).wait()

        @pl.when(i > 0)
        def _():
            pltpu.make_async_copy(out_v.at[1],
                                  out_hbm.at[pl.ds(0, 8), 0], semo1).wait()

        shuffle(1)
        fire_in(u0 + 3, 1, semi1)
        fire_out(u0 + 1, 1, semo1)
        return carry

    lax.fori_loop(0, UPW // 2, pair_body, 0)
    pltpu.make_async_copy(tex4_hbm.at[:, 0, 0], in_v.at[0], semi0).wait()
    pltpu.make_async_copy(tex4_hbm.at[:, 0, 0], in_v.at[1], semi1).wait()
    pltpu.make_async_copy(out_v.at[0], out_hbm.at[pl.ds(0, 8), 0], semo0).wait()
    pltpu.make_async_copy(out_v.at[1], out_hbm.at[pl.ds(0, 8), 0], semo1).wait()


P = 512
SUBS = 8
NSUP = NPTS // NW // (SUBS * P)
NGRP = SUBS * P // 16


def _gs_body(uv_hbm, tex_hbm, out_hbm, u_v, v_v, idx_v, gA, gB,
             w_v, ob_v, semu, semA, semB, semo):
    w = _wid()
    lane = lax.iota(jnp.int32, 16)
    zero = jnp.zeros((16,), jnp.int32)

    def fire_gathers(sub, buf, sem):
        cps = []
        for k in range(4):
            for m in range(4):
                cps.append(pltpu.async_copy(
                    tex_hbm.at[idx_v.at[sub * 16 + k * 4 + m]],
                    buf.at[k, pl.ds(m * 128, 128)], sem))
        return cps

    def drain_gathers(buf, sem):
        for k in range(4):
            pltpu.make_async_copy(
                tex_hbm.at[pl.ds(0, P)], buf.at[k], sem).wait()

    def super_body(s, carry):
        rtg = w * NSUP + s
        b = rtg // 64
        rt = rtg % 64
        i0 = rt * 8

        cps = []
        for k in range(4):
            cps.append(pltpu.async_copy(
                uv_hbm.at[b, pl.ds(i0, 8), k, 0], u_v.at[k], semu))
            cps.append(pltpu.async_copy(
                uv_hbm.at[b, pl.ds(i0, 8), k, 1], v_v.at[k], semu))
        for cp in cps:
            cp.wait()

        @pl.when(s > 0)
        def _():
            for _i in range(32):
                pltpu.make_async_copy(
                    uv_hbm.at[0, pl.ds(0, 8), 0, 0],
                    ob_v.at[0, :, pl.ds(0, 128)], semo).wait()

        def index_body(j, carry):
            sub = j // 32
            jj = j % 32
            k4 = (j // 8) % 4
            col = (j % 8) * 16
            u = u_v[k4, sub, pl.ds(col, 16)]
            v = v_v[k4, sub, pl.ds(col, 16)]
            fu = u - u.astype(jnp.int32).astype(jnp.float32)
            fv = v - v.astype(jnp.int32).astype(jnp.float32)
            x = ((fu * 2.0 - 1.0) + 1.0) * 0.5 * float(RES - 1)
            y = ((fv * 2.0 - 1.0) + 1.0) * 0.5 * float(RES - 1)
            xi = x.astype(jnp.int32)
            yi = y.astype(jnp.int32)
            wx = x - xi.astype(jnp.float32)
            wy = y - yi.astype(jnp.float32)
            i00 = yi * RES + xi
            m = jj // 8
            o = (jj % 8) * 16
            r0 = sub * 16 + m
            idx_v[r0, pl.ds(o, 16)] = i00
            idx_v[r0 + 4, pl.ds(o, 16)] = i00 + 1
            idx_v[r0 + 8, pl.ds(o, 16)] = i00 + RES
            idx_v[r0 + 12, pl.ds(o, 16)] = i00 + RES + 1
            o16 = j * 16
            w_v[0, pl.ds(o16, 16)] = wx
            w_v[1, pl.ds(o16, 16)] = wy
            return carry

        lax.fori_loop(0, NGRP, index_body, 0, unroll=2)

        fire_gathers(0, gA, semA)
        fire_gathers(1, gB, semB)
        for sub in range(SUBS):
            buf, sem = (gA, semA) if sub % 2 == 0 else (gB, semB)
            drain_gathers(buf, sem)

            def blend_body(j2, pidx8, sub=sub, buf=buf):
                o16 = j2 * 16
                wbase = sub * P + o16
                wx = w_v[0, pl.ds(wbase, 16)]
                wy = w_v[1, pl.ds(wbase, 16)]
                mx = 1.0 - wx
                my = 1.0 - wy
                w00 = mx * my
                w01 = wx * my
                w10 = mx * wy
                w11 = wx * wy
                for c in range(CH):
                    ci = pidx8 + c
                    v00 = plsc.load_gather(buf.at[0], [zero, ci])
                    v01 = plsc.load_gather(buf.at[1], [zero, ci])
                    v10 = plsc.load_gather(buf.at[2], [zero, ci])
                    v11 = plsc.load_gather(buf.at[3], [zero, ci])
                    ob_v[c, sub, pl.ds(o16, 16)] = (w00 * v00 + w01 * v01
                                                    + w10 * v10 + w11 * v11)
                return pidx8 + 128

            lax.fori_loop(0, P // 16, blend_body, lane * CH, unroll=2)
            if sub + 2 < SUBS:
                fire_gathers(sub + 2, buf, sem)

        for c in range(CH):
            for ct in range(4):
                pltpu.async_copy(ob_v.at[c, :, pl.ds(ct * 128, 128)],
                                 out_hbm.at[b, c, rt, ct], semo)
        return carry

    lax.fori_loop(0, NSUP, super_body, 0)
    for _i in range(32):
        pltpu.make_async_copy(uv_hbm.at[0, pl.ds(0, 8), 0, 0],
                              ob_v.at[0, :, pl.ds(0, 128)], semo).wait()


_CACHE = {}


def _build():
    if "fns" not in _CACHE:
        mesh = plsc.VectorSubcoreMesh(core_axis_name="c", subcore_axis_name="s")
        tr = pl.kernel(
            _tr_body,
            out_type=jax.ShapeDtypeStruct((RES, 16, 1024), jnp.float32),
            mesh=mesh,
            scratch_types=[
                pltpu.VMEM((2, CH, 1024), jnp.float32),
                pltpu.VMEM((2, 8, 1024), jnp.float32),
                pltpu.SemaphoreType.DMA,
                pltpu.SemaphoreType.DMA,
                pltpu.SemaphoreType.DMA,
                pltpu.SemaphoreType.DMA,
            ],
            compiler_params=_PARAMS)
        gs = pl.kernel(
            _gs_body,
            out_type=jax.ShapeDtypeStruct((B, CH, 64, 4, 8, 128), jnp.float32),
            mesh=mesh,
            scratch_types=[
                pltpu.VMEM((4, 8, 128), jnp.float32),
                pltpu.VMEM((4, 8, 128), jnp.float32),
                pltpu.VMEM((128, 128), jnp.int32),
                pltpu.VMEM((4, P, CH), jnp.float32),
                pltpu.VMEM((4, P, CH), jnp.float32),
                pltpu.VMEM((2, SUBS * P), jnp.float32),
                pltpu.VMEM((CH, SUBS, P), jnp.float32),
                pltpu.SemaphoreType.DMA,
                pltpu.SemaphoreType.DMA,
                pltpu.SemaphoreType.DMA,
                pltpu.SemaphoreType.DMA,
            ],
            compiler_params=_PARAMS)
        _CACHE["fns"] = (tr, gs)
    return _CACHE["fns"]


def kernel(uvs, tex):
    tr, gs = _build()
    tex4 = (tex.reshape(CH, 256, 8, 16, 128).transpose(0, 1, 3, 2, 4)
            .reshape(CH, 256, 16, 1024))
    uv_phys = uvs.reshape(B, 512, 4, 128, 2).transpose(0, 1, 2, 4, 3)
    tex_t = tr(tex4).reshape(RES * RES, CH)
    out6 = gs(uv_phys, tex_t)
    return out6.transpose(0, 1, 2, 4, 3, 5).reshape(B, CH, 512, 512)


# speedup vs baseline: 1.0243x; 1.0243x over previous
import jax
import jax.numpy as jnp
from jax import lax
from jax.experimental import pallas as pl
from jax.experimental.pallas import tpu as pltpu
from jax.experimental.pallas import tpu_sc as plsc

RES = 2048
CH = 8
B = 4
HW = 512 * 512
NPTS = B * HW
NW = 32

_PARAMS = pltpu.CompilerParams(needs_layout_passes=False,
                               use_tc_tiling_on_sc=False)


def _wid():
    return lax.axis_index("s") * 2 + lax.axis_index("c")


NUNIT = 256 * 8
UPW = NUNIT // NW
UW = 2048


def _tr_body(tex4_hbm, out_hbm, in_v, out_v, semi0, semi1, semi2, semi3,
             semo0, semo1):
    w = _wid()
    lane = lax.iota(jnp.int32, 16)
    cvec = lane % 8
    dvec = lane // 8
    zero = jnp.zeros((16,), jnp.int32)
    flat0 = cvec * UW + dvec
    semis = (semi0, semi1, semi2, semi3)
    semos = (semo0, semo1)

    def fire_in(u, p):
        uid = jnp.minimum(w * UPW + u, NUNIT - 1)
        yt = uid // 8
        x2 = uid % 8
        return pltpu.async_copy(
            tex4_hbm.at[:, yt, x2], in_v.at[p], semis[p])

    def shuffle(p):
        for xtl in range(2):
            def shuf(h, idxv, p=p, xtl=xtl):
                val = plsc.load_gather(in_v.at[p], [zero, idxv])
                out_v[p % 2, h // 64,
                      pl.ds(xtl * 1024 + (h % 64) * 16, 16)] = val
                return idxv + 2

            lax.fori_loop(0, 512, shuf, flat0 + xtl * 1024, unroll=8)

    def fire_out(u, p):
        uid = w * UPW + u
        yt = uid // 8
        x2 = uid % 8
        return pltpu.async_copy(
            out_v.at[p % 2], out_hbm.at[pl.ds(yt * 8, 8), x2], semos[p % 2])

    for p in range(4):
        fire_in(p, p)

    def quad_body(i, carry):
        u0 = 4 * i
        for p in range(4):
            pltpu.make_async_copy(
                tex4_hbm.at[:, 0, 0], in_v.at[p], semis[p]).wait()

            if p < 2:
                @pl.when(i > 0)
                def _(p=p):
                    pltpu.make_async_copy(
                        out_v.at[p % 2], out_hbm.at[pl.ds(0, 8), 0],
                        semos[p % 2]).wait()
            else:
                pltpu.make_async_copy(
                    out_v.at[p % 2], out_hbm.at[pl.ds(0, 8), 0],
                    semos[p % 2]).wait()

            shuffle(p)
            fire_in(u0 + p + 4, p)
            fire_out(u0 + p, p)
        return carry

    lax.fori_loop(0, UPW // 4, quad_body, 0)
    for p in range(4):
        pltpu.make_async_copy(
            tex4_hbm.at[:, 0, 0], in_v.at[p], semis[p]).wait()
    for p in range(2):
        pltpu.make_async_copy(
            out_v.at[p], out_hbm.at[pl.ds(0, 8), 0], semos[p]).wait()


P = 512
SUBS = 8
NSUP = NPTS // NW // (SUBS * P)


def _gs_body(uv_hbm, tex_hbm, out_hbm, u_v, v_v, idx_v, gA, gB,
             w_v, ob_v, semu, semA, semB, semo):
    w = _wid()
    lane = lax.iota(jnp.int32, 16)
    zero = jnp.zeros((16,), jnp.int32)

    def fire_uv(rtg):
        rtg = jnp.minimum(rtg, 255)
        b = rtg // 64
        i0 = (rtg % 64) * 8
        for k in range(4):
            pltpu.async_copy(uv_hbm.at[b, pl.ds(i0, 8), k, 0],
                             u_v.at[k], semu)
            pltpu.async_copy(uv_hbm.at[b, pl.ds(i0, 8), k, 1],
                             v_v.at[k], semu)

    def drain_uv():
        for k in range(4):
            pltpu.make_async_copy(uv_hbm.at[0, pl.ds(0, 8), 0, 0],
                                  u_v.at[k], semu).wait()
            pltpu.make_async_copy(uv_hbm.at[0, pl.ds(0, 8), 0, 0],
                                  v_v.at[k], semu).wait()

    def index_sub(sub):
        def index_body(jj, carry, sub=sub):
            k4 = (jj // 8) % 4
            col = (jj % 8) * 16
            u = u_v[k4, sub, pl.ds(col, 16)]
            v = v_v[k4, sub, pl.ds(col, 16)]
            fu = u - u.astype(jnp.int32).astype(jnp.float32)
            fv = v - v.astype(jnp.int32).astype(jnp.float32)
            x = ((fu * 2.0 - 1.0) + 1.0) * 0.5 * float(RES - 1)
            y = ((fv * 2.0 - 1.0) + 1.0) * 0.5 * float(RES - 1)
            xi = x.astype(jnp.int32)
            yi = y.astype(jnp.int32)
            wx = x - xi.astype(jnp.float32)
            wy = y - yi.astype(jnp.float32)
            i00 = yi * RES + xi
            m = jj // 8
            o = (jj % 8) * 16
            r0 = sub * 16 + m
            idx_v[r0, pl.ds(o, 16)] = i00
            idx_v[r0 + 4, pl.ds(o, 16)] = i00 + 1
            idx_v[r0 + 8, pl.ds(o, 16)] = i00 + RES
            idx_v[r0 + 12, pl.ds(o, 16)] = i00 + RES + 1
            w_v[0, pl.ds(sub * P + jj * 16, 16)] = wx
            w_v[1, pl.ds(sub * P + jj * 16, 16)] = wy
            return carry

        lax.fori_loop(0, P // 16, index_body, 0, unroll=2)

    def fire_gathers(sub, buf, sem):
        for k in range(4):
            for m in range(4):
                pltpu.async_copy(
                    tex_hbm.at[idx_v.at[sub * 16 + k * 4 + m]],
                    buf.at[k, pl.ds(m * 128, 128)], sem)

    def drain_gathers(buf, sem):
        for k in range(4):
            pltpu.make_async_copy(
                tex_hbm.at[pl.ds(0, P)], buf.at[k], sem).wait()

    def super_body(s, carry):
        rtg = w * NSUP + s
        b = rtg // 64
        rt = rtg % 64

        drain_uv()
        index_sub(0)
        fire_gathers(0, gA, semA)
        index_sub(1)
        fire_gathers(1, gB, semB)

        @pl.when(s > 0)
        def _():
            for _i in range(32):
                pltpu.make_async_copy(
                    uv_hbm.at[0, pl.ds(0, 8), 0, 0],
                    ob_v.at[0, :, pl.ds(0, 128)], semo).wait()

        for sub in range(SUBS):
            buf, sem = (gA, semA) if sub % 2 == 0 else (gB, semB)
            if sub + 2 < SUBS:
                index_sub(sub + 2)
            drain_gathers(buf, sem)
            if sub == SUBS - 1:
                fire_uv(rtg + 1)

            def blend_body(j2, pidx8, sub=sub, buf=buf):
                o16 = j2 * 16
                wbase = sub * P + o16
                wx = w_v[0, pl.ds(wbase, 16)]
                wy = w_v[1, pl.ds(wbase, 16)]
                mx = 1.0 - wx
                my = 1.0 - wy
                w00 = mx * my
                w01 = wx * my
                w10 = mx * wy
                w11 = wx * wy
                for c in range(CH):
                    ci = pidx8 + c
                    v00 = plsc.load_gather(buf.at[0], [zero, ci])
                    v01 = plsc.load_gather(buf.at[1], [zero, ci])
                    v10 = plsc.load_gather(buf.at[2], [zero, ci])
                    v11 = plsc.load_gather(buf.at[3], [zero, ci])
                    ob_v[c, sub, pl.ds(o16, 16)] = (w00 * v00 + w01 * v01
                                                    + w10 * v10 + w11 * v11)
                return pidx8 + 128

            lax.fori_loop(0, P // 16, blend_body, lane * CH, unroll=2)
            if sub + 2 < SUBS:
                fire_gathers(sub + 2, buf, sem)

        for c in range(CH):
            for ct in range(4):
                pltpu.async_copy(ob_v.at[c, :, pl.ds(ct * 128, 128)],
                                 out_hbm.at[b, c, rt, ct], semo)
        return carry

    fire_uv(w * NSUP)
    lax.fori_loop(0, NSUP, super_body, 0)
    drain_uv()
    for _i in range(32):
        pltpu.make_async_copy(uv_hbm.at[0, pl.ds(0, 8), 0, 0],
                              ob_v.at[0, :, pl.ds(0, 128)], semo).wait()


_CACHE = {}


def _build():
    if "fns" not in _CACHE:
        mesh = plsc.VectorSubcoreMesh(core_axis_name="c", subcore_axis_name="s")
        tr = pl.kernel(
            _tr_body,
            out_type=jax.ShapeDtypeStruct((RES, 8, 2048), jnp.float32),
            mesh=mesh,
            scratch_types=[
                pltpu.VMEM((4, CH, UW), jnp.float32),
                pltpu.VMEM((2, 8, 2048), jnp.float32),
                pltpu.SemaphoreType.DMA,
                pltpu.SemaphoreType.DMA,
                pltpu.SemaphoreType.DMA,
                pltpu.SemaphoreType.DMA,
                pltpu.SemaphoreType.DMA,
                pltpu.SemaphoreType.DMA,
            ],
            compiler_params=_PARAMS)
        gs = pl.kernel(
            _gs_body,
            out_type=jax.ShapeDtypeStruct((B, CH, 64, 4, 8, 128), jnp.float32),
            mesh=mesh,
            scratch_types=[
                pltpu.VMEM((4, 8, 128), jnp.float32),
                pltpu.VMEM((4, 8, 128), jnp.float32),
                pltpu.VMEM((128, 128), jnp.int32),
                pltpu.VMEM((4, P, CH), jnp.float32),
                pltpu.VMEM((4, P, CH), jnp.float32),
                pltpu.VMEM((2, SUBS * P), jnp.float32),
                pltpu.VMEM((CH, SUBS, P), jnp.float32),
                pltpu.SemaphoreType.DMA,
                pltpu.SemaphoreType.DMA,
                pltpu.SemaphoreType.DMA,
                pltpu.SemaphoreType.DMA,
            ],
            compiler_params=_PARAMS)
        _CACHE["fns"] = (tr, gs)
    return _CACHE["fns"]


def kernel(uvs, tex):
    tr, gs = _build()
    tex4 = (tex.reshape(CH, 256, 8, 16, 128).transpose(0, 1, 3, 2, 4)
            .reshape(CH, 256, 8, 2048))
    uv_phys = uvs.reshape(B, 512, 4, 128, 2).transpose(0, 1, 2, 4, 3)
    tex_t = tr(tex4).reshape(RES * RES, CH)
    out6 = gs(uv_phys, tex_t)
    return out6.transpose(0, 1, 2, 4, 3, 5).reshape(B, CH, 512, 512)


# speedup vs baseline: 1.3880x; 1.3551x over previous
import jax
import jax.numpy as jnp
from jax import lax
from jax.experimental import pallas as pl
from jax.experimental.pallas import tpu as pltpu
from jax.experimental.pallas import tpu_sc as plsc

RES = 2048
CH = 8
B = 4
HW = 512 * 512
NPTS = B * HW
NW = 32

_PARAMS = pltpu.CompilerParams(needs_layout_passes=False,
                               use_tc_tiling_on_sc=False)


def _wid():
    return lax.axis_index("s") * 2 + lax.axis_index("c")


NUNIT = 256 * 8
UPW = NUNIT // NW
UW = 2048


def _tr_body(tex4_hbm, out_hbm, in_v, out_v, semi0, semi1, semi2, semi3,
             semo0, semo1):
    w = _wid()
    lane = lax.iota(jnp.int32, 16)
    cvec = lane % 8
    dvec = lane // 8
    zero = jnp.zeros((16,), jnp.int32)
    flat0 = cvec * 2050 + dvec
    semis = (semi0, semi1, semi2, semi3)
    semos = (semo0, semo1)

    def fire_in(u, p):
        uid = jnp.minimum(w * UPW + u, NUNIT - 1)
        yt = uid // 8
        x2 = uid % 8
        return pltpu.async_copy(
            tex4_hbm.at[:, yt, x2], in_v.at[p, :, pl.ds(0, UW)], semis[p])

    def shuffle(p):
        for xtl in range(2):
            def shuf(h, idxv, p=p, xtl=xtl):
                val = plsc.load_gather(in_v.at[p], [zero, idxv])
                out_v[p % 2, h // 64,
                      pl.ds(xtl * 1024 + (h % 64) * 16, 16)] = val
                return idxv + 2

            lax.fori_loop(0, 512, shuf, flat0 + xtl * 1024, unroll=8)

    def fire_out(u, p):
        uid = w * UPW + u
        yt = uid // 8
        x2 = uid % 8
        return pltpu.async_copy(
            out_v.at[p % 2], out_hbm.at[pl.ds(yt * 8, 8), x2], semos[p % 2])

    for p in range(4):
        fire_in(p, p)

    def quad_body(i, carry):
        u0 = 4 * i
        for p in range(4):
            pltpu.make_async_copy(
                tex4_hbm.at[:, 0, 0], in_v.at[p, :, pl.ds(0, UW)],
                semis[p]).wait()

            if p < 2:
                @pl.when(i > 0)
                def _(p=p):
                    pltpu.make_async_copy(
                        out_v.at[p % 2], out_hbm.at[pl.ds(0, 8), 0],
                        semos[p % 2]).wait()
            else:
                pltpu.make_async_copy(
                    out_v.at[p % 2], out_hbm.at[pl.ds(0, 8), 0],
                    semos[p % 2]).wait()

            shuffle(p)
            fire_in(u0 + p + 4, p)
            fire_out(u0 + p, p)
        return carry

    lax.fori_loop(0, UPW // 4, quad_body, 0)
    for p in range(4):
        pltpu.make_async_copy(
            tex4_hbm.at[:, 0, 0], in_v.at[p, :, pl.ds(0, UW)],
            semis[p]).wait()
    for p in range(2):
        pltpu.make_async_copy(
            out_v.at[p], out_hbm.at[pl.ds(0, 8), 0], semos[p]).wait()


P = 512
SUBS = 8
NSUP = NPTS // NW // (SUBS * P)


def _gs_body(uv_hbm, tex_hbm, out_hbm, u_v, v_v, idx_v, gA, gB,
             w_v, ob_v, semu, semA, semB, semo):
    w = _wid()
    lane = lax.iota(jnp.int32, 16)
    zero = jnp.zeros((16,), jnp.int32)

    def fire_uv(rtg):
        rtg = jnp.minimum(rtg, 255)
        b = rtg // 64
        i0 = (rtg % 64) * 8
        for k in range(4):
            pltpu.async_copy(uv_hbm.at[b, pl.ds(i0, 8), k, 0],
                             u_v.at[k], semu)
            pltpu.async_copy(uv_hbm.at[b, pl.ds(i0, 8), k, 1],
                             v_v.at[k], semu)

    def drain_uv():
        for k in range(4):
            pltpu.make_async_copy(uv_hbm.at[0, pl.ds(0, 8), 0, 0],
                                  u_v.at[k], semu).wait()
            pltpu.make_async_copy(uv_hbm.at[0, pl.ds(0, 8), 0, 0],
                                  v_v.at[k], semu).wait()

    def index_sub(sub):
        def index_body(jj, carry, sub=sub):
            k4 = (jj // 8) % 4
            col = (jj % 8) * 16
            u = u_v[k4, sub, pl.ds(col, 16)]
            v = v_v[k4, sub, pl.ds(col, 16)]
            fu = u - u.astype(jnp.int32).astype(jnp.float32)
            fv = v - v.astype(jnp.int32).astype(jnp.float32)
            x = ((fu * 2.0 - 1.0) + 1.0) * 0.5 * float(RES - 1)
            y = ((fv * 2.0 - 1.0) + 1.0) * 0.5 * float(RES - 1)
            xi = x.astype(jnp.int32)
            yi = y.astype(jnp.int32)
            wx = x - xi.astype(jnp.float32)
            wy = y - yi.astype(jnp.float32)
            i00 = yi * RES + xi
            m = jj // 8
            o = (jj % 8) * 16
            r0 = sub * 16 + m
            idx_v[r0, pl.ds(o, 16)] = i00
            idx_v[r0 + 4, pl.ds(o, 16)] = i00 + 1
            idx_v[r0 + 8, pl.ds(o, 16)] = i00 + RES
            idx_v[r0 + 12, pl.ds(o, 16)] = i00 + RES + 1
            w_v[0, pl.ds(sub * P + jj * 16, 16)] = wx
            w_v[1, pl.ds(sub * P + jj * 16, 16)] = wy
            return carry

        lax.fori_loop(0, P // 16, index_body, 0, unroll=2)

    def fire_gathers(sub, buf, sem):
        for k in range(4):
            for m in range(4):
                pltpu.async_copy(
                    tex_hbm.at[idx_v.at[sub * 16 + k * 4 + m]],
                    buf.at[k, pl.ds(m * 128, 128)], sem)

    def drain_gathers(buf, sem):
        for k in range(4):
            pltpu.make_async_copy(
                tex_hbm.at[pl.ds(0, P)], buf.at[k], sem).wait()

    def super_body(s, carry):
        rtg = w * NSUP + s
        b = rtg // 64
        rt = rtg % 64

        drain_uv()
        index_sub(0)
        fire_gathers(0, gA, semA)
        index_sub(1)
        fire_gathers(1, gB, semB)

        @pl.when(s > 0)
        def _():
            for _i in range(32):
                pltpu.make_async_copy(
                    uv_hbm.at[0, pl.ds(0, 8), 0, 0],
                    ob_v.at[0, :, pl.ds(0, 128)], semo).wait()

        for sub in range(SUBS):
            buf, sem = (gA, semA) if sub % 2 == 0 else (gB, semB)
            if sub + 2 < SUBS:
                index_sub(sub + 2)
            drain_gathers(buf, sem)
            if sub == SUBS - 1:
                fire_uv(rtg + 1)

            def blend_body(j2, pidx8, sub=sub, buf=buf):
                o16 = j2 * 16
                wbase = sub * P + o16
                wx = w_v[0, pl.ds(wbase, 16)]
                wy = w_v[1, pl.ds(wbase, 16)]
                mx = 1.0 - wx
                my = 1.0 - wy
                w00 = mx * my
                w01 = wx * my
                w10 = mx * wy
                w11 = wx * wy
                for c in range(CH):
                    ci = pidx8 + c
                    v00 = plsc.load_gather(buf.at[0], [zero, ci])
                    v01 = plsc.load_gather(buf.at[1], [zero, ci])
                    v10 = plsc.load_gather(buf.at[2], [zero, ci])
                    v11 = plsc.load_gather(buf.at[3], [zero, ci])
                    ob_v[c, sub, pl.ds(o16, 16)] = (w00 * v00 + w01 * v01
                                                    + w10 * v10 + w11 * v11)
                return pidx8 + 16 * CH

            lax.fori_loop(0, P // 16, blend_body, lane * CH, unroll=2)
            if sub + 2 < SUBS:
                fire_gathers(sub + 2, buf, sem)

        for c in range(CH):
            for ct in range(4):
                pltpu.async_copy(ob_v.at[c, :, pl.ds(ct * 128, 128)],
                                 out_hbm.at[b, c, rt, ct], semo)
        return carry

    fire_uv(w * NSUP)
    lax.fori_loop(0, NSUP, super_body, 0)
    drain_uv()
    for _i in range(32):
        pltpu.make_async_copy(uv_hbm.at[0, pl.ds(0, 8), 0, 0],
                              ob_v.at[0, :, pl.ds(0, 128)], semo).wait()


_CACHE = {}


def _build():
    if "fns" not in _CACHE:
        mesh = plsc.VectorSubcoreMesh(core_axis_name="c", subcore_axis_name="s")
        tr = pl.kernel(
            _tr_body,
            out_type=jax.ShapeDtypeStruct((RES, 8, 2048), jnp.float32),
            mesh=mesh,
            scratch_types=[
                pltpu.VMEM((4, CH, 2050), jnp.float32),
                pltpu.VMEM((2, 8, 2048), jnp.float32),
                pltpu.SemaphoreType.DMA,
                pltpu.SemaphoreType.DMA,
                pltpu.SemaphoreType.DMA,
                pltpu.SemaphoreType.DMA,
                pltpu.SemaphoreType.DMA,
                pltpu.SemaphoreType.DMA,
            ],
            compiler_params=_PARAMS)
        gs = pl.kernel(
            _gs_body,
            out_type=jax.ShapeDtypeStruct((B, CH, 64, 4, 8, 128), jnp.float32),
            mesh=mesh,
            scratch_types=[
                pltpu.VMEM((4, 8, 128), jnp.float32),
                pltpu.VMEM((4, 8, 128), jnp.float32),
                pltpu.VMEM((128, 128), jnp.int32),
                pltpu.VMEM((4, P, CH), jnp.float32),
                pltpu.VMEM((4, P, CH), jnp.float32),
                pltpu.VMEM((2, SUBS * P), jnp.float32),
                pltpu.VMEM((CH, SUBS, P), jnp.float32),
                pltpu.SemaphoreType.DMA,
                pltpu.SemaphoreType.DMA,
                pltpu.SemaphoreType.DMA,
                pltpu.SemaphoreType.DMA,
            ],
            compiler_params=_PARAMS)
        _CACHE["fns"] = (tr, gs)
    return _CACHE["fns"]


def kernel(uvs, tex):
    tr, gs = _build()
    tex4 = (tex.reshape(CH, 256, 8, 16, 128).transpose(0, 1, 3, 2, 4)
            .reshape(CH, 256, 8, 2048))
    uv_phys = uvs.reshape(B, 512, 4, 128, 2).transpose(0, 1, 2, 4, 3)
    tex_t = tr(tex4).reshape(RES * RES, CH)
    out6 = gs(uv_phys, tex_t)
    return out6.transpose(0, 1, 2, 4, 3, 5).reshape(B, CH, 512, 512)
